# baseline (speedup 1.0000x reference)
import functools
import math

import jax
import jax.numpy as jnp
from jax.experimental import pallas as pl
from jax.experimental.pallas import tpu as pltpu
from jax.sharding import PartitionSpec as P

_REG = 0.05
_MAX_ITER = 100
_NEG_INF = -1e30

_DOT_DIMS = (((1,), (1,)), ((), ()))


def _norms_kernel(x_ref, y_ref, x2_ref, y2_ref):
    x = x_ref[...]
    y = y_ref[...]
    x2_ref[...] = jnp.sum(x * x, axis=1, keepdims=True)
    y2_ref[...] = jnp.sum(y * y, axis=1, keepdims=True)


def _norms(x, y):
    n, _ = x.shape
    m, _ = y.shape
    return pl.pallas_call(
        _norms_kernel,
        out_shape=(
            jax.ShapeDtypeStruct((n, 1), jnp.float32),
            jax.ShapeDtypeStruct((m, 1), jnp.float32),
        ),
        name="sqnorms",
    )(x, y)


def _dot3(x, y):
    xh = x.astype(jnp.bfloat16)
    xl = (x - xh.astype(jnp.float32)).astype(jnp.bfloat16)
    yh = y.astype(jnp.bfloat16)
    yl = (y - yh.astype(jnp.float32)).astype(jnp.bfloat16)

    def d(a, b):
        return jax.lax.dot_general(a, b, _DOT_DIMS,
                                   preferred_element_type=jnp.float32)

    return d(xh, yh) + (d(xh, yl) + d(xl, yh))


def _mr_kernel(x_ref, y_ref, x2_ref, y2_ref, mr_ref):
    xy = _dot3(x_ref[...], y_ref[...])
    d2 = (x2_ref[...] + y2_ref[...]) - 2.0 * xy
    mr_ref[...] = jnp.maximum(d2, 0.0) * jnp.float32(-1.0 / _REG)


def _mr_call(x, y, x2, y2):
    n, k = x.shape
    m = y.shape[0]
    it = min(1024, n)
    jt = min(1024, m)
    return pl.pallas_call(
        _mr_kernel,
        grid=(n // it, m // jt),
        in_specs=[
            pl.BlockSpec((it, k), lambda a, b: (a, 0)),
            pl.BlockSpec((jt, k), lambda a, b: (b, 0)),
            pl.BlockSpec((it, 1), lambda a, b: (a, 0)),
            pl.BlockSpec((1, jt), lambda a, b: (0, b)),
        ],
        out_specs=pl.BlockSpec((it, jt), lambda a, b: (a, b)),
        out_shape=jax.ShapeDtypeStruct((n, m), jnp.float32),
        compiler_params=pltpu.CompilerParams(
            dimension_semantics=("parallel", "arbitrary")),
        name="mr_matrix",
    )(x, y, x2, y2)


def _u_from_partials(loga, mall, sall):
    mx = jnp.max(mall, axis=1, keepdims=True)
    ssum = jnp.sum(sall * jnp.exp(mall - mx), axis=1, keepdims=True)
    return loga - (jnp.log(ssum) + mx)


def _v_kernel(nblk_i, loga, logb, mr_ref, mall_ref, sall_ref, v_ref, m_ref, s_ref):
    i = pl.program_id(1)
    u = _u_from_partials(loga, mall_ref[...], sall_ref[...])
    t = mr_ref[...] + u
    tmax = jnp.max(t, axis=0, keepdims=True)
    tsum = jnp.sum(jnp.exp(t - tmax), axis=0, keepdims=True)

    @pl.when(i == 0)
    def _():
        m_ref[...] = jnp.full_like(m_ref, _NEG_INF)
        s_ref[...] = jnp.zeros_like(s_ref)

    m_old = m_ref[...]
    m_new = jnp.maximum(m_old, tmax)
    s_ref[...] = s_ref[...] * jnp.exp(m_old - m_new) + tsum * jnp.exp(tmax - m_new)
    m_ref[...] = m_new

    @pl.when(i == nblk_i - 1)
    def _():
        v_ref[...] = logb - (jnp.log(s_ref[...]) + m_ref[...])


def _pass_v(mr, mall, sall, loga, logb):
    n, m = mr.shape
    ncols = mall.shape[1]
    it = min(1024, n)
    jt = min(2048, m)
    ni = n // it
    return pl.pallas_call(
        lambda *refs: _v_kernel(ni, loga, logb, *refs),
        grid=(m // jt, ni),
        in_specs=[
            pl.BlockSpec((it, jt), lambda a, b: (b, a)),
            pl.BlockSpec((it, ncols), lambda a, b: (b, 0)),
            pl.BlockSpec((it, ncols), lambda a, b: (b, 0)),
        ],
        out_specs=pl.BlockSpec((1, jt), lambda a, b: (0, a)),
        out_shape=jax.ShapeDtypeStruct((1, m), jnp.float32),
        scratch_shapes=[pltpu.VMEM((1, jt), jnp.float32),
                        pltpu.VMEM((1, jt), jnp.float32)],
        compiler_params=pltpu.CompilerParams(
            dimension_semantics=("parallel", "arbitrary")),
        name="sinkhorn_v",
    )(mr, mall, sall)


def _u_kernel(nblk_j, mr_ref, v_ref, mout_ref, sout_ref, m_ref, s_ref):
    j = pl.program_id(1)
    t = mr_ref[...] + v_ref[...]
    tmax = jnp.max(t, axis=1, keepdims=True)
    tsum = jnp.sum(jnp.exp(t - tmax), axis=1, keepdims=True)

    @pl.when(j == 0)
    def _():
        m_ref[...] = jnp.full_like(m_ref, _NEG_INF)
        s_ref[...] = jnp.zeros_like(s_ref)

    m_old = m_ref[...]
    m_new = jnp.maximum(m_old, tmax)
    s_ref[...] = s_ref[...] * jnp.exp(m_old - m_new) + tsum * jnp.exp(tmax - m_new)
    m_ref[...] = m_new

    @pl.when(j == nblk_j - 1)
    def _():
        mout_ref[...] = m_ref[...]
        sout_ref[...] = s_ref[...]


def _pass_u_partial(mr, v):
    n, m = mr.shape
    it = min(2048, n)
    jt = min(1024, m)
    nj = m // jt
    return pl.pallas_call(
        lambda *refs: _u_kernel(nj, *refs),
        grid=(n // it, nj),
        in_specs=[
            pl.BlockSpec((it, jt), lambda a, b: (a, b)),
            pl.BlockSpec((1, jt), lambda a, b: (0, b)),
        ],
        out_specs=[pl.BlockSpec((it, 1), lambda a, b: (a, 0)),
                   pl.BlockSpec((it, 1), lambda a, b: (a, 0))],
        out_shape=[jax.ShapeDtypeStruct((n, 1), jnp.float32),
                   jax.ShapeDtypeStruct((n, 1), jnp.float32)],
        scratch_shapes=[pltpu.VMEM((it, 1), jnp.float32),
                        pltpu.VMEM((it, 1), jnp.float32)],
        compiler_params=pltpu.CompilerParams(
            dimension_semantics=("parallel", "arbitrary")),
        name="sinkhorn_u",
    )(mr, v)


def _loss_kernel(loga, mr_ref, mall_ref, sall_ref, v_ref, o_ref):
    u = _u_from_partials(loga, mall_ref[...], sall_ref[...])
    mr = mr_ref[...]
    contrib = jnp.exp(mr + u + v_ref[...]) * mr
    psum = jnp.sum(contrib, axis=0, keepdims=True)
    o_ref[...] = psum.reshape(o_ref.shape)


def _loss_call(mr, mall, sall, v, loga):
    n, m = mr.shape
    ncols = mall.shape[1]
    it = min(1024, n)
    jt = min(2048, m)
    ni = n // it
    return pl.pallas_call(
        lambda *refs: _loss_kernel(loga, *refs),
        grid=(ni, m // jt),
        in_specs=[
            pl.BlockSpec((it, jt), lambda a, b: (a, b)),
            pl.BlockSpec((it, ncols), lambda a, b: (a, 0)),
            pl.BlockSpec((it, ncols), lambda a, b: (a, 0)),
            pl.BlockSpec((1, jt), lambda a, b: (0, b)),
        ],
        out_specs=pl.BlockSpec((1, 1, jt), lambda a, b: (a, 0, b)),
        out_shape=jax.ShapeDtypeStruct((ni, 1, m), jnp.float32),
        compiler_params=pltpu.CompilerParams(
            dimension_semantics=("parallel", "arbitrary")),
        name="sinkhorn_loss",
    )(mr, mall, sall, v)


def _sinkhorn_local(x, y, csize, c_axis):
    n, _ = x.shape
    m, _ = y.shape
    m_loc = m // csize
    loga = float(-math.log(float(n)))
    logb = float(-math.log(float(m)))
    cc = jax.lax.axis_index(c_axis)

    x2, y2c = _norms(x, y)
    y2 = y2c.reshape(1, m)

    y_h = jax.lax.dynamic_slice_in_dim(y, cc * m_loc, m_loc, 0)
    y2_h = jax.lax.dynamic_slice_in_dim(y2, cc * m_loc, m_loc, 1)

    mr_cols = _mr_call(x, y_h, x2, y2_h)

    def body(_, carry):
        mall, sall, _v = carry
        v_h = _pass_v(mr_cols, mall, sall, loga, logb)
        m_h, s_h = _pass_u_partial(mr_cols, v_h)
        mall = jax.lax.all_gather(m_h, c_axis, axis=1, tiled=True)
        sall = jax.lax.all_gather(s_h, c_axis, axis=1, tiled=True)
        return (mall, sall, v_h)

    mall0 = jnp.full((n, csize), loga, jnp.float32)
    sall0 = jnp.full((n, csize), 1.0 / csize, jnp.float32)
    v0 = jnp.zeros((1, m_loc), jnp.float32)
    mall, sall, v_h = jax.lax.fori_loop(0, _MAX_ITER, body, (mall0, sall0, v0))

    partials = _loss_call(mr_cols, mall, sall, v_h, loga)
    return jax.lax.psum(jnp.sum(partials), c_axis)


def kernel(x, y):
    x = x.astype(jnp.float32)
    y = y.astype(jnp.float32)
    n, _ = x.shape
    m, _ = y.shape
    ndev = jax.device_count()
    csize = 2 if (ndev >= 2 and n % 2 == 0 and m % 2 == 0) else 1
    mesh = jax.make_mesh((csize,), ("c",))
    fn = jax.shard_map(
        functools.partial(_sinkhorn_local, csize=csize, c_axis="c"),
        mesh=mesh,
        in_specs=(P(None, None), P(None, None)),
        out_specs=P(),
        check_vma=False,
    )
    return fn(x, y) * jnp.float32(-_REG)


# speedup vs baseline: 1.0846x; 1.0846x over previous
import functools
import math

import jax
import jax.numpy as jnp
from jax.experimental import pallas as pl
from jax.experimental.pallas import tpu as pltpu
from jax.sharding import PartitionSpec as P

_REG = 0.05
_MAX_ITER = 100
_NEG_INF = -1e30

_DOT_DIMS = (((1,), (1,)), ((), ()))


def _norms_kernel(x_ref, y_ref, x2_ref, y2_ref):
    x = x_ref[...]
    y = y_ref[...]
    x2_ref[...] = jnp.sum(x * x, axis=1, keepdims=True)
    y2_ref[...] = jnp.sum(y * y, axis=1, keepdims=True)


def _norms(x, y):
    n, _ = x.shape
    m, _ = y.shape
    return pl.pallas_call(
        _norms_kernel,
        out_shape=(
            jax.ShapeDtypeStruct((n, 1), jnp.float32),
            jax.ShapeDtypeStruct((m, 1), jnp.float32),
        ),
        name="sqnorms",
    )(x, y)


def _dot3(x, y):
    xh = x.astype(jnp.bfloat16)
    xl = (x - xh.astype(jnp.float32)).astype(jnp.bfloat16)
    yh = y.astype(jnp.bfloat16)
    yl = (y - yh.astype(jnp.float32)).astype(jnp.bfloat16)

    def d(a, b):
        return jax.lax.dot_general(a, b, _DOT_DIMS,
                                   preferred_element_type=jnp.float32)

    return d(xh, yh) + (d(xh, yl) + d(xl, yh))


def _mr_kernel(x_ref, y_ref, x2_ref, y2_ref, mr_ref):
    xy = _dot3(x_ref[...], y_ref[...])
    d2 = (x2_ref[...] + y2_ref[...]) - 2.0 * xy
    mr_ref[...] = jnp.maximum(d2, 0.0) * jnp.float32(-1.0 / _REG)


def _mr_call(x, y, x2, y2):
    n, k = x.shape
    m = y.shape[0]
    it = min(1024, n)
    jt = min(1024, m)
    return pl.pallas_call(
        _mr_kernel,
        grid=(n // it, m // jt),
        in_specs=[
            pl.BlockSpec((it, k), lambda a, b: (a, 0)),
            pl.BlockSpec((jt, k), lambda a, b: (b, 0)),
            pl.BlockSpec((it, 1), lambda a, b: (a, 0)),
            pl.BlockSpec((1, jt), lambda a, b: (0, b)),
        ],
        out_specs=pl.BlockSpec((it, jt), lambda a, b: (a, b)),
        out_shape=jax.ShapeDtypeStruct((n, m), jnp.float32),
        compiler_params=pltpu.CompilerParams(
            dimension_semantics=("parallel", "arbitrary")),
        name="mr_matrix",
    )(x, y, x2, y2)


def _v_kernel(nblk_i, logb, mr_ref, u_ref, v_ref, m_ref, s_ref):
    i = pl.program_id(1)
    t = mr_ref[...] + u_ref[...]
    tmax = jnp.max(t, axis=0, keepdims=True)
    tsum = jnp.sum(jnp.exp(t - tmax), axis=0, keepdims=True)

    @pl.when(i == 0)
    def _():
        m_ref[...] = jnp.full_like(m_ref, _NEG_INF)
        s_ref[...] = jnp.zeros_like(s_ref)

    m_old = m_ref[...]
    m_new = jnp.maximum(m_old, tmax)
    s_ref[...] = s_ref[...] * jnp.exp(m_old - m_new) + tsum * jnp.exp(tmax - m_new)
    m_ref[...] = m_new

    @pl.when(i == nblk_i - 1)
    def _():
        v_ref[...] = logb - (jnp.log(s_ref[...]) + m_ref[...])


def _pass_v(mr, u, logb):
    n, m = mr.shape
    it = min(1024, n)
    jt = min(2048, m)
    ni = n // it
    return pl.pallas_call(
        lambda *refs: _v_kernel(ni, logb, *refs),
        grid=(m // jt, ni),
        in_specs=[
            pl.BlockSpec((it, jt), lambda a, b: (b, a)),
            pl.BlockSpec((it, 1), lambda a, b: (b, 0)),
        ],
        out_specs=pl.BlockSpec((1, jt), lambda a, b: (0, a)),
        out_shape=jax.ShapeDtypeStruct((1, m), jnp.float32),
        scratch_shapes=[pltpu.VMEM((1, jt), jnp.float32),
                        pltpu.VMEM((1, jt), jnp.float32)],
        compiler_params=pltpu.CompilerParams(
            dimension_semantics=("parallel", "arbitrary")),
        name="sinkhorn_v",
    )(mr, u)


def _u_kernel(nblk_j, loga, mr_ref, v_ref, u_ref, m_ref, s_ref):
    j = pl.program_id(1)
    t = mr_ref[...] + v_ref[...]
    tmax = jnp.max(t, axis=1, keepdims=True)
    tsum = jnp.sum(jnp.exp(t - tmax), axis=1, keepdims=True)

    @pl.when(j == 0)
    def _():
        m_ref[...] = jnp.full_like(m_ref, _NEG_INF)
        s_ref[...] = jnp.zeros_like(s_ref)

    m_old = m_ref[...]
    m_new = jnp.maximum(m_old, tmax)
    s_ref[...] = s_ref[...] * jnp.exp(m_old - m_new) + tsum * jnp.exp(tmax - m_new)
    m_ref[...] = m_new

    @pl.when(j == nblk_j - 1)
    def _():
        u_ref[...] = loga - (jnp.log(s_ref[...]) + m_ref[...])


def _pass_u(mr, v, loga):
    n, m = mr.shape
    it = min(2048, n)
    jt = min(1024, m)
    nj = m // jt
    return pl.pallas_call(
        lambda *refs: _u_kernel(nj, loga, *refs),
        grid=(n // it, nj),
        in_specs=[
            pl.BlockSpec((it, jt), lambda a, b: (a, b)),
            pl.BlockSpec((1, jt), lambda a, b: (0, b)),
        ],
        out_specs=pl.BlockSpec((it, 1), lambda a, b: (a, 0)),
        out_shape=jax.ShapeDtypeStruct((n, 1), jnp.float32),
        scratch_shapes=[pltpu.VMEM((it, 1), jnp.float32),
                        pltpu.VMEM((it, 1), jnp.float32)],
        compiler_params=pltpu.CompilerParams(
            dimension_semantics=("parallel", "arbitrary")),
        name="sinkhorn_u",
    )(mr, v)


def _loss_kernel(mr_ref, u_ref, v_ref, o_ref):
    mr = mr_ref[...]
    contrib = jnp.exp(mr + u_ref[...] + v_ref[...]) * mr
    psum = jnp.sum(contrib, axis=0, keepdims=True)
    o_ref[...] = psum.reshape(o_ref.shape)


def _loss_call(mr, u, v):
    n, m = mr.shape
    it = min(1024, n)
    jt = min(2048, m)
    ni = n // it
    return pl.pallas_call(
        _loss_kernel,
        grid=(ni, m // jt),
        in_specs=[
            pl.BlockSpec((it, jt), lambda a, b: (a, b)),
            pl.BlockSpec((it, 1), lambda a, b: (a, 0)),
            pl.BlockSpec((1, jt), lambda a, b: (0, b)),
        ],
        out_specs=pl.BlockSpec((1, 1, jt), lambda a, b: (a, 0, b)),
        out_shape=jax.ShapeDtypeStruct((ni, 1, m), jnp.float32),
        compiler_params=pltpu.CompilerParams(
            dimension_semantics=("parallel", "arbitrary")),
        name="sinkhorn_loss",
    )(mr, u, v)


def _sinkhorn_local(x, y, csize, c_axis):
    n, _ = x.shape
    m, _ = y.shape
    n_loc = n // csize
    m_loc = m // csize
    loga = float(-math.log(float(n)))
    logb = float(-math.log(float(m)))
    cc = jax.lax.axis_index(c_axis)

    x2, y2c = _norms(x, y)
    y2 = y2c.reshape(1, m)

    x_h = jax.lax.dynamic_slice_in_dim(x, cc * n_loc, n_loc, 0)
    y_h = jax.lax.dynamic_slice_in_dim(y, cc * m_loc, m_loc, 0)
    x2_h = jax.lax.dynamic_slice_in_dim(x2, cc * n_loc, n_loc, 0)
    y2_h = jax.lax.dynamic_slice_in_dim(y2, cc * m_loc, m_loc, 1)

    mr_cols = _mr_call(x, y_h, x2, y2_h)
    mr_rows = _mr_call(x_h, y, x2_h, y2)

    def body(_, uv):
        u, v = uv
        v_h = _pass_v(mr_cols, u, logb)
        v = jax.lax.all_gather(v_h, c_axis, axis=1, tiled=True)
        u_h = _pass_u(mr_rows, v, loga)
        u = jax.lax.all_gather(u_h, c_axis, axis=0, tiled=True)
        return (u, v)

    u0 = jnp.zeros((n, 1), jnp.float32)
    v0 = jnp.zeros((1, m), jnp.float32)
    u, v = jax.lax.fori_loop(0, _MAX_ITER, body, (u0, v0))

    u_h = jax.lax.dynamic_slice_in_dim(u, cc * n_loc, n_loc, 0)
    partials = _loss_call(mr_rows, u_h, v)
    return jax.lax.psum(jnp.sum(partials), c_axis)


def kernel(x, y):
    x = x.astype(jnp.float32)
    y = y.astype(jnp.float32)
    n, _ = x.shape
    m, _ = y.shape
    ndev = jax.device_count()
    csize = 2 if (ndev >= 2 and n % 2 == 0 and m % 2 == 0) else 1
    mesh = jax.make_mesh((csize,), ("c",))
    fn = jax.shard_map(
        functools.partial(_sinkhorn_local, csize=csize, c_axis="c"),
        mesh=mesh,
        in_specs=(P(None, None), P(None, None)),
        out_specs=P(),
        check_vma=False,
    )
    return fn(x, y) * jnp.float32(-_REG)


# speedup vs baseline: 1.2189x; 1.1238x over previous
import functools
import math

import jax
import jax.numpy as jnp
from jax.experimental import pallas as pl
from jax.experimental.pallas import tpu as pltpu
from jax.sharding import PartitionSpec as P

_REG = 0.05
_MAX_ITER = 100
_NEG_INF = -1e30

_DOT_DIMS = (((1,), (1,)), ((), ()))


def _norms_kernel(x_ref, y_ref, x2_ref, y2_ref):
    x = x_ref[...]
    y = y_ref[...]
    x2_ref[...] = jnp.sum(x * x, axis=1, keepdims=True)
    y2_ref[...] = jnp.sum(y * y, axis=1, keepdims=True)


def _norms(x, y):
    n, _ = x.shape
    m, _ = y.shape
    return pl.pallas_call(
        _norms_kernel,
        out_shape=(
            jax.ShapeDtypeStruct((n, 1), jnp.float32),
            jax.ShapeDtypeStruct((m, 1), jnp.float32),
        ),
        name="sqnorms",
    )(x, y)


def _dot3(x, y):
    xh = x.astype(jnp.bfloat16)
    xl = (x - xh.astype(jnp.float32)).astype(jnp.bfloat16)
    yh = y.astype(jnp.bfloat16)
    yl = (y - yh.astype(jnp.float32)).astype(jnp.bfloat16)

    def d(a, b):
        return jax.lax.dot_general(a, b, _DOT_DIMS,
                                   preferred_element_type=jnp.float32)

    return d(xh, yh) + (d(xh, yl) + d(xl, yh))


def _mr_kernel(x_ref, y_ref, x2_ref, y2_ref, mr_ref):
    xy = _dot3(x_ref[...], y_ref[...])
    d2 = (x2_ref[...] + y2_ref[...]) - 2.0 * xy
    mr_ref[...] = jnp.maximum(d2, 0.0) * jnp.float32(-1.0 / _REG)


def _mr_call(x, y, x2, y2):
    n, k = x.shape
    m = y.shape[0]
    it = min(1024, n)
    jt = min(1024, m)
    return pl.pallas_call(
        _mr_kernel,
        grid=(n // it, m // jt),
        in_specs=[
            pl.BlockSpec((it, k), lambda a, b: (a, 0)),
            pl.BlockSpec((jt, k), lambda a, b: (b, 0)),
            pl.BlockSpec((it, 1), lambda a, b: (a, 0)),
            pl.BlockSpec((1, jt), lambda a, b: (0, b)),
        ],
        out_specs=pl.BlockSpec((it, jt), lambda a, b: (a, b)),
        out_shape=jax.ShapeDtypeStruct((n, m), jnp.float32),
        compiler_params=pltpu.CompilerParams(
            dimension_semantics=("parallel", "arbitrary")),
        name="mr_matrix",
    )(x, y, x2, y2)


def _v_from_partials(csize, logb, p):
    mx = p[0:1, :]
    for c in range(1, csize):
        mx = jnp.maximum(mx, p[2 * c:2 * c + 1, :])
    ssum = p[1:2, :] * jnp.exp(p[0:1, :] - mx)
    for c in range(1, csize):
        ssum = ssum + p[2 * c + 1:2 * c + 2, :] * jnp.exp(p[2 * c:2 * c + 1, :] - mx)
    return logb - (jnp.log(ssum) + mx)


def _v_kernel(nblk_i, mr_ref, u_ref, p_ref, m_ref, s_ref):
    i = pl.program_id(1)
    t = mr_ref[...] + u_ref[...]
    tmax = jnp.max(t, axis=0, keepdims=True)
    tsum = jnp.sum(jnp.exp(t - tmax), axis=0, keepdims=True)

    @pl.when(i == 0)
    def _():
        m_ref[...] = jnp.full_like(m_ref, _NEG_INF)
        s_ref[...] = jnp.zeros_like(s_ref)

    m_old = m_ref[...]
    m_new = jnp.maximum(m_old, tmax)
    s_ref[...] = s_ref[...] * jnp.exp(m_old - m_new) + tsum * jnp.exp(tmax - m_new)
    m_ref[...] = m_new

    @pl.when(i == nblk_i - 1)
    def _():
        p_ref[0:1, :] = m_ref[...]
        p_ref[1:2, :] = s_ref[...]


def _pass_v_partial(mr, u):
    n, m = mr.shape
    it = min(1024, n)
    jt = min(2048, m)
    ni = n // it
    return pl.pallas_call(
        lambda *refs: _v_kernel(ni, *refs),
        grid=(m // jt, ni),
        in_specs=[
            pl.BlockSpec((it, jt), lambda a, b: (b, a)),
            pl.BlockSpec((it, 1), lambda a, b: (b, 0)),
        ],
        out_specs=pl.BlockSpec((2, jt), lambda a, b: (0, a)),
        out_shape=jax.ShapeDtypeStruct((2, m), jnp.float32),
        scratch_shapes=[pltpu.VMEM((1, jt), jnp.float32),
                        pltpu.VMEM((1, jt), jnp.float32)],
        compiler_params=pltpu.CompilerParams(
            dimension_semantics=("parallel", "arbitrary")),
        name="sinkhorn_v",
    )(mr, u)


def _u_kernel(nblk_j, csize, loga, logb, mr_ref, p_ref, u_ref, m_ref, s_ref):
    j = pl.program_id(1)
    v = _v_from_partials(csize, logb, p_ref[...])
    t = mr_ref[...] + v
    tmax = jnp.max(t, axis=1, keepdims=True)
    tsum = jnp.sum(jnp.exp(t - tmax), axis=1, keepdims=True)

    @pl.when(j == 0)
    def _():
        m_ref[...] = jnp.full_like(m_ref, _NEG_INF)
        s_ref[...] = jnp.zeros_like(s_ref)

    m_old = m_ref[...]
    m_new = jnp.maximum(m_old, tmax)
    s_ref[...] = s_ref[...] * jnp.exp(m_old - m_new) + tsum * jnp.exp(tmax - m_new)
    m_ref[...] = m_new

    @pl.when(j == nblk_j - 1)
    def _():
        u_ref[...] = loga - (jnp.log(s_ref[...]) + m_ref[...])


def _pass_u(mr, pall, csize, loga, logb):
    n, m = mr.shape
    prows = pall.shape[0]
    it = min(2048, n)
    jt = min(1024, m)
    nj = m // jt
    return pl.pallas_call(
        lambda *refs: _u_kernel(nj, csize, loga, logb, *refs),
        grid=(n // it, nj),
        in_specs=[
            pl.BlockSpec((it, jt), lambda a, b: (a, b)),
            pl.BlockSpec((prows, jt), lambda a, b: (0, b)),
        ],
        out_specs=pl.BlockSpec((it, 1), lambda a, b: (a, 0)),
        out_shape=jax.ShapeDtypeStruct((n, 1), jnp.float32),
        scratch_shapes=[pltpu.VMEM((it, 1), jnp.float32),
                        pltpu.VMEM((it, 1), jnp.float32)],
        compiler_params=pltpu.CompilerParams(
            dimension_semantics=("parallel", "arbitrary")),
        name="sinkhorn_u",
    )(mr, pall)


def _loss_kernel(csize, loga, logb, mr_ref, u_ref, p_ref, o_ref):
    v = _v_from_partials(csize, logb, p_ref[...])
    mr = mr_ref[...]
    contrib = jnp.exp(mr + u_ref[...] + v) * mr
    psum = jnp.sum(contrib, axis=0, keepdims=True)
    o_ref[...] = psum.reshape(o_ref.shape)


def _loss_call(mr, u, pall, csize, loga, logb):
    n, m = mr.shape
    prows = pall.shape[0]
    it = min(1024, n)
    jt = min(2048, m)
    ni = n // it
    return pl.pallas_call(
        lambda *refs: _loss_kernel(csize, loga, logb, *refs),
        grid=(ni, m // jt),
        in_specs=[
            pl.BlockSpec((it, jt), lambda a, b: (a, b)),
            pl.BlockSpec((it, 1), lambda a, b: (a, 0)),
            pl.BlockSpec((prows, jt), lambda a, b: (0, b)),
        ],
        out_specs=pl.BlockSpec((1, 1, jt), lambda a, b: (a, 0, b)),
        out_shape=jax.ShapeDtypeStruct((ni, 1, m), jnp.float32),
        compiler_params=pltpu.CompilerParams(
            dimension_semantics=("parallel", "arbitrary")),
        name="sinkhorn_loss",
    )(mr, u, pall)


def _sinkhorn_local(x, y, csize, c_axis):
    n, _ = x.shape
    m, _ = y.shape
    n_loc = n // csize
    loga = float(-math.log(float(n)))
    logb = float(-math.log(float(m)))
    cc = jax.lax.axis_index(c_axis)

    x2, y2c = _norms(x, y)
    y2 = y2c.reshape(1, m)

    x_h = jax.lax.dynamic_slice_in_dim(x, cc * n_loc, n_loc, 0)
    x2_h = jax.lax.dynamic_slice_in_dim(x2, cc * n_loc, n_loc, 0)

    mr_rows = _mr_call(x_h, y, x2_h, y2)

    def body(_, carry):
        u_h, _pall = carry
        p_h = _pass_v_partial(mr_rows, u_h)
        pall = jax.lax.all_gather(p_h, c_axis, axis=0, tiled=True)
        u_h = _pass_u(mr_rows, pall, csize, loga, logb)
        return (u_h, pall)

    u0 = jnp.zeros((n_loc, 1), jnp.float32)
    pall0 = jnp.zeros((2 * csize, m), jnp.float32)
    u_h, pall = jax.lax.fori_loop(0, _MAX_ITER, body, (u0, pall0))

    partials = _loss_call(mr_rows, u_h, pall, csize, loga, logb)
    return jax.lax.psum(jnp.sum(partials), c_axis)


def kernel(x, y):
    x = x.astype(jnp.float32)
    y = y.astype(jnp.float32)
    n, _ = x.shape
    m, _ = y.shape
    ndev = jax.device_count()
    csize = 2 if (ndev >= 2 and n % 2 == 0 and m % 2 == 0) else 1
    mesh = jax.make_mesh((csize,), ("c",))
    fn = jax.shard_map(
        functools.partial(_sinkhorn_local, csize=csize, c_axis="c"),
        mesh=mesh,
        in_specs=(P(None, None), P(None, None)),
        out_specs=P(),
        check_vma=False,
    )
    return fn(x, y) * jnp.float32(-_REG)


# speedup vs baseline: 1.2418x; 1.0188x over previous
import functools
import math

import jax
import jax.numpy as jnp
from jax.experimental import pallas as pl
from jax.experimental.pallas import tpu as pltpu
from jax.sharding import PartitionSpec as P

_REG = 0.05
_MAX_ITER = 100
_NEG_INF = -1e30

_DOT_DIMS = (((1,), (1,)), ((), ()))


def _norms_kernel(x_ref, y_ref, x2_ref, y2_ref):
    x = x_ref[...]
    y = y_ref[...]
    x2_ref[...] = jnp.sum(x * x, axis=1, keepdims=True)
    y2_ref[...] = jnp.sum(y * y, axis=1, keepdims=True)


def _norms(x, y):
    n, _ = x.shape
    m, _ = y.shape
    return pl.pallas_call(
        _norms_kernel,
        out_shape=(
            jax.ShapeDtypeStruct((n, 1), jnp.float32),
            jax.ShapeDtypeStruct((m, 1), jnp.float32),
        ),
        name="sqnorms",
    )(x, y)


def _dot3(x, y):
    xh = x.astype(jnp.bfloat16)
    xl = (x - xh.astype(jnp.float32)).astype(jnp.bfloat16)
    yh = y.astype(jnp.bfloat16)
    yl = (y - yh.astype(jnp.float32)).astype(jnp.bfloat16)

    def d(a, b):
        return jax.lax.dot_general(a, b, _DOT_DIMS,
                                   preferred_element_type=jnp.float32)

    return d(xh, yh) + (d(xh, yl) + d(xl, yh))


def _mr_kernel(x_ref, y_ref, x2_ref, y2_ref, mr_ref):
    xy = _dot3(x_ref[...], y_ref[...])
    d2 = (x2_ref[...] + y2_ref[...]) - 2.0 * xy
    mr_ref[...] = jnp.maximum(d2, 0.0) * jnp.float32(-1.0 / _REG)


def _mr_call(x, y, x2, y2):
    n, k = x.shape
    m = y.shape[0]
    it = min(1024, n)
    jt = min(1024, m)
    return pl.pallas_call(
        _mr_kernel,
        grid=(n // it, m // jt),
        in_specs=[
            pl.BlockSpec((it, k), lambda a, b: (a, 0)),
            pl.BlockSpec((jt, k), lambda a, b: (b, 0)),
            pl.BlockSpec((it, 1), lambda a, b: (a, 0)),
            pl.BlockSpec((1, jt), lambda a, b: (0, b)),
        ],
        out_specs=pl.BlockSpec((it, jt), lambda a, b: (a, b)),
        out_shape=jax.ShapeDtypeStruct((n, m), jnp.float32),
        compiler_params=pltpu.CompilerParams(
            dimension_semantics=("parallel", "arbitrary")),
        name="mr_matrix",
    )(x, y, x2, y2)


def _v_from_partials(csize, logb, p):
    mx = p[0:1, :]
    for c in range(1, csize):
        mx = jnp.maximum(mx, p[2 * c:2 * c + 1, :])
    ssum = p[1:2, :] * jnp.exp(p[0:1, :] - mx)
    for c in range(1, csize):
        ssum = ssum + p[2 * c + 1:2 * c + 2, :] * jnp.exp(p[2 * c:2 * c + 1, :] - mx)
    return logb - (jnp.log(ssum) + mx)


def _v_kernel(nblk_i, mr_ref, u_ref, p_ref, m_ref, s_ref):
    i = pl.program_id(1)
    t = mr_ref[...] + u_ref[...]
    tmax = jnp.max(t, axis=0, keepdims=True)
    tsum = jnp.sum(jnp.exp(t - tmax), axis=0, keepdims=True)

    @pl.when(i == 0)
    def _():
        m_ref[...] = jnp.full_like(m_ref, _NEG_INF)
        s_ref[...] = jnp.zeros_like(s_ref)

    m_old = m_ref[...]
    m_new = jnp.maximum(m_old, tmax)
    s_ref[...] = s_ref[...] * jnp.exp(m_old - m_new) + tsum * jnp.exp(tmax - m_new)
    m_ref[...] = m_new

    @pl.when(i == nblk_i - 1)
    def _():
        p_ref[0:1, :] = m_ref[...]
        p_ref[1:2, :] = s_ref[...]


def _pass_v_partial(mr, u):
    n, m = mr.shape
    it = min(1024, n)
    jt = min(4096, m)
    ni = n // it
    return pl.pallas_call(
        lambda *refs: _v_kernel(ni, *refs),
        grid=(m // jt, ni),
        in_specs=[
            pl.BlockSpec((it, jt), lambda a, b: (b, a)),
            pl.BlockSpec((it, 1), lambda a, b: (b, 0)),
        ],
        out_specs=pl.BlockSpec((2, jt), lambda a, b: (0, a)),
        out_shape=jax.ShapeDtypeStruct((2, m), jnp.float32),
        scratch_shapes=[pltpu.VMEM((1, jt), jnp.float32),
                        pltpu.VMEM((1, jt), jnp.float32)],
        compiler_params=pltpu.CompilerParams(
            dimension_semantics=("parallel", "arbitrary")),
        name="sinkhorn_v",
    )(mr, u)


def _u_kernel(nblk_j, csize, loga, logb, mr_ref, p_ref, u_ref, m_ref, s_ref):
    j = pl.program_id(1)
    v = _v_from_partials(csize, logb, p_ref[...])
    t = mr_ref[...] + v
    tmax = jnp.max(t, axis=1, keepdims=True)
    tsum = jnp.sum(jnp.exp(t - tmax), axis=1, keepdims=True)

    @pl.when(j == 0)
    def _():
        m_ref[...] = jnp.full_like(m_ref, _NEG_INF)
        s_ref[...] = jnp.zeros_like(s_ref)

    m_old = m_ref[...]
    m_new = jnp.maximum(m_old, tmax)
    s_ref[...] = s_ref[...] * jnp.exp(m_old - m_new) + tsum * jnp.exp(tmax - m_new)
    m_ref[...] = m_new

    @pl.when(j == nblk_j - 1)
    def _():
        u_ref[...] = loga - (jnp.log(s_ref[...]) + m_ref[...])


def _pass_u(mr, pall, csize, loga, logb):
    n, m = mr.shape
    prows = pall.shape[0]
    it = min(2048, n)
    jt = min(2048, m)
    nj = m // jt
    return pl.pallas_call(
        lambda *refs: _u_kernel(nj, csize, loga, logb, *refs),
        grid=(n // it, nj),
        in_specs=[
            pl.BlockSpec((it, jt), lambda a, b: (a, b)),
            pl.BlockSpec((prows, jt), lambda a, b: (0, b)),
        ],
        out_specs=pl.BlockSpec((it, 1), lambda a, b: (a, 0)),
        out_shape=jax.ShapeDtypeStruct((n, 1), jnp.float32),
        scratch_shapes=[pltpu.VMEM((it, 1), jnp.float32),
                        pltpu.VMEM((it, 1), jnp.float32)],
        compiler_params=pltpu.CompilerParams(
            dimension_semantics=("parallel", "arbitrary")),
        name="sinkhorn_u",
    )(mr, pall)


def _loss_kernel(csize, loga, logb, mr_ref, u_ref, p_ref, o_ref):
    v = _v_from_partials(csize, logb, p_ref[...])
    mr = mr_ref[...]
    contrib = jnp.exp(mr + u_ref[...] + v) * mr
    psum = jnp.sum(contrib, axis=0, keepdims=True)
    o_ref[...] = psum.reshape(o_ref.shape)


def _loss_call(mr, u, pall, csize, loga, logb):
    n, m = mr.shape
    prows = pall.shape[0]
    it = min(1024, n)
    jt = min(2048, m)
    ni = n // it
    return pl.pallas_call(
        lambda *refs: _loss_kernel(csize, loga, logb, *refs),
        grid=(ni, m // jt),
        in_specs=[
            pl.BlockSpec((it, jt), lambda a, b: (a, b)),
            pl.BlockSpec((it, 1), lambda a, b: (a, 0)),
            pl.BlockSpec((prows, jt), lambda a, b: (0, b)),
        ],
        out_specs=pl.BlockSpec((1, 1, jt), lambda a, b: (a, 0, b)),
        out_shape=jax.ShapeDtypeStruct((ni, 1, m), jnp.float32),
        compiler_params=pltpu.CompilerParams(
            dimension_semantics=("parallel", "arbitrary")),
        name="sinkhorn_loss",
    )(mr, u, pall)


def _sinkhorn_local(x, y, csize, c_axis):
    n, _ = x.shape
    m, _ = y.shape
    n_loc = n // csize
    loga = float(-math.log(float(n)))
    logb = float(-math.log(float(m)))
    cc = jax.lax.axis_index(c_axis)

    x2, y2c = _norms(x, y)
    y2 = y2c.reshape(1, m)

    x_h = jax.lax.dynamic_slice_in_dim(x, cc * n_loc, n_loc, 0)
    x2_h = jax.lax.dynamic_slice_in_dim(x2, cc * n_loc, n_loc, 0)

    mr_rows = _mr_call(x_h, y, x2_h, y2)

    def body(_, carry):
        u_h, _pall = carry
        p_h = _pass_v_partial(mr_rows, u_h)
        pall = jax.lax.all_gather(p_h, c_axis, axis=0, tiled=True)
        u_h = _pass_u(mr_rows, pall, csize, loga, logb)
        return (u_h, pall)

    u0 = jnp.zeros((n_loc, 1), jnp.float32)
    pall0 = jnp.zeros((2 * csize, m), jnp.float32)
    u_h, pall = jax.lax.fori_loop(0, _MAX_ITER, body, (u0, pall0))

    partials = _loss_call(mr_rows, u_h, pall, csize, loga, logb)
    return jax.lax.psum(jnp.sum(partials), c_axis)


def kernel(x, y):
    x = x.astype(jnp.float32)
    y = y.astype(jnp.float32)
    n, _ = x.shape
    m, _ = y.shape
    ndev = jax.device_count()
    csize = 2 if (ndev >= 2 and n % 2 == 0 and m % 2 == 0) else 1
    mesh = jax.make_mesh((csize,), ("c",))
    fn = jax.shard_map(
        functools.partial(_sinkhorn_local, csize=csize, c_axis="c"),
        mesh=mesh,
        in_specs=(P(None, None), P(None, None)),
        out_specs=P(),
        check_vma=False,
    )
    return fn(x, y) * jnp.float32(-_REG)
